# Initial kernel scaffold; baseline (speedup 1.0000x reference)
import jax
import jax.numpy as jnp
from jax.experimental import pallas as pl


def kernel(logits, target):
    raise NotImplementedError("write your pallas kernel here")


# speedup vs baseline: 1.2383x; 1.2383x over previous
import functools

import jax
import jax.numpy as jnp
from jax import lax
from jax.experimental import pallas as pl

BATCH = 16384
NCLS = 1000
SAVE = 8192
BLOCK_ROWS = 1024
NBLK = BATCH // BLOCK_ROWS


def _hard_mining_kernel(x_ref, tgt_ref, out_ref, loss_ref):
    i = pl.program_id(0)
    x = x_ref[...]
    m = jnp.max(x, axis=-1, keepdims=True)
    s = jnp.sum(jnp.exp(x - m), axis=-1, keepdims=True)
    lse = jnp.log(s) + m
    tgt = tgt_ref[0, 0, :]
    col = lax.broadcasted_iota(jnp.int32, (BLOCK_ROWS, NCLS), 1)
    xt = jnp.sum(jnp.where(col == tgt[:, None], x, 0.0), axis=-1)
    loss_ref[i, :] = lse[:, 0] - xt

    @pl.when(i == NBLK - 1)
    def _select():
        losses = loss_ref[...]
        bits = lax.bitcast_convert_type(losses, jnp.int32)

        def body(b, t):
            cand = t | (jnp.int32(1) << b)
            cnt = jnp.sum((bits >= cand).astype(jnp.int32))
            return jnp.where(cnt >= SAVE, cand, t)

        t_bits = lax.fori_loop(0, 31, lambda j, t: body(30 - j, t),
                               jnp.int32(0))
        t = lax.bitcast_convert_type(t_bits, jnp.float32)
        gt = losses > t
        n_gt = jnp.sum(gt.astype(jnp.int32))
        s_gt = jnp.sum(jnp.where(gt, losses, 0.0))
        mean = (s_gt + (SAVE - n_gt).astype(jnp.float32) * t) / SAVE
        out_ref[...] = mean.reshape(1, 1)


@jax.jit
def _run(logits, target):
    tgt3 = target.astype(jnp.int32).reshape(NBLK, 1, BLOCK_ROWS)
    out = pl.pallas_call(
        _hard_mining_kernel,
        grid=(NBLK,),
        in_specs=[
            pl.BlockSpec((BLOCK_ROWS, NCLS), lambda i: (i, 0)),
            pl.BlockSpec((1, 1, BLOCK_ROWS), lambda i: (i, 0, 0)),
        ],
        out_specs=pl.BlockSpec((1, 1), lambda i: (0, 0)),
        out_shape=jax.ShapeDtypeStruct((1, 1), jnp.float32),
        scratch_shapes=[pltpu_vmem((NBLK, BLOCK_ROWS), jnp.float32)],
    )(logits, tgt3)
    return out[0, 0]


def pltpu_vmem(shape, dtype):
    from jax.experimental.pallas import tpu as pltpu
    return pltpu.VMEM(shape, dtype)


def kernel(logits, target):
    return _run(logits, target)
